# baseline (speedup 1.0000x reference)
import functools

import jax
import jax.numpy as jnp
from jax import lax
from jax.experimental import pallas as pl
from jax.experimental.pallas import tpu as pltpu
from jax.experimental.pallas import tpu_sc as plsc

N = 5000
D = 256
B = 128
NB = 40
NP = NB * B
IOU_THRESH = 0.5
SCORE_THRESH = 0.2

_SC_NC = 2
_SC_NS = 16
_SC_NW = _SC_NC * _SC_NS
_ROWS_PER_W = NP // _SC_NW


def _nms_body(rows_ref, cols_ref, s_ref, gated_ref, keep_ref, coff_ref,
              mf_ref):
    offf = jnp.max(rows_ref[:, 0:4]) + 1.0
    labc = cols_ref[4:5, :]
    offc = labc * offf
    x1c = cols_ref[0:1, :] + offc
    y1c = cols_ref[1:2, :] + offc
    x2c = cols_ref[2:3, :] + offc
    y2c = cols_ref[3:4, :] + offc
    coff_ref[0:1, :] = x1c
    coff_ref[1:2, :] = y1c
    coff_ref[2:3, :] = x2c
    coff_ref[3:4, :] = y2c
    area_c = (x2c - x1c) * (y2c - y1c)
    coff_ref[4:5, :] = area_c

    keep_ref[...] = jnp.ones((NB, B), jnp.float32)

    n_above = jnp.sum((s_ref[...] >= SCORE_THRESH).astype(jnp.int32))
    nblk = (n_above + (B - 1)) // B

    tri_r = lax.broadcasted_iota(jnp.int32, (B, B), 0)
    tri_c = lax.broadcasted_iota(jnp.int32, (B, B), 1)
    trif = (tri_r < tri_c).astype(jnp.float32)

    def make_blk_body(cs, nbw):
        c0 = cs * B
        cw = nbw * B
        xc1 = coff_ref[0:1, c0:c0 + cw]
        yc1 = coff_ref[1:2, c0:c0 + cw]
        xc2 = coff_ref[2:3, c0:c0 + cw]
        yc2 = coff_ref[3:4, c0:c0 + cw]
        ac = coff_ref[4:5, c0:c0 + cw]
        blk_row = lax.broadcasted_iota(jnp.int32, (nbw, B), 0) + cs

        def blk_body(k, _):
            base = pl.multiple_of(k * B, B)
            r = rows_ref[pl.ds(base, B), :]
            offr = r[:, 4:5] * offf
            x1r = r[:, 0:1] + offr
            y1r = r[:, 1:2] + offr
            x2r = r[:, 2:3] + offr
            y2r = r[:, 3:4] + offr
            area_r = (x2r - x1r) * (y2r - y1r)

            def strip_body(sI, _):
                rs = rows_ref[pl.ds(base + sI * 8, 8), :]
                offs = rs[:, 4:5] * offf
                x1s = rs[:, 0:1] + offs
                y1s = rs[:, 1:2] + offs
                x2s = rs[:, 2:3] + offs
                y2s = rs[:, 3:4] + offs
                area_s = (x2s - x1s) * (y2s - y1s)
                iw = jnp.maximum(jnp.minimum(x2s, xc2) - jnp.maximum(x1s, xc1),
                                 0.0)
                ih = jnp.minimum(y2s, yc2) - jnp.maximum(y1s, yc1)
                inter = iw * ih
                union = area_s + ac - inter
                mf_ref[pl.ds(sI * 8, 8), 0:cw] = (
                    inter > IOU_THRESH * union).astype(jnp.float32)
                return 0

            lax.fori_loop(0, B // 8, strip_body, 0)

            x1b = coff_ref[0:1, pl.ds(base, B)]
            y1b = coff_ref[1:2, pl.ds(base, B)]
            x2b = coff_ref[2:3, pl.ds(base, B)]
            y2b = coff_ref[3:4, pl.ds(base, B)]
            area_b = coff_ref[4:5, pl.ds(base, B)]
            iwb = jnp.maximum(jnp.minimum(x2r, x2b) - jnp.maximum(x1r, x1b), 0.0)
            ihb = jnp.minimum(y2r, y2b) - jnp.maximum(y1r, y1b)
            interb = iwb * ihb
            unionb = area_r + area_b - interb
            mbf = (interb > IOU_THRESH * unionb).astype(jnp.float32) * trif

            init = keep_ref[pl.ds(k, 1), :]

            def step(kb):
                supp = lax.dot_general(kb, mbf, (((1,), (0,)), ((), ())),
                                       preferred_element_type=jnp.float32)
                return init * (supp < 0.5).astype(jnp.float32)

            def wcond(c):
                return c[1]

            def wbody(c):
                kb, _ = c
                kb1 = step(kb)
                kb2 = step(kb1)
                return (kb2, jnp.any(kb2 != kb1))

            kb, _ = lax.while_loop(wcond, wbody, (init, True))
            keep_ref[pl.ds(k, 1), :] = kb

            supp_all = lax.dot_general(kb, mf_ref[:, 0:cw],
                                       (((1,), (0,)), ((), ())),
                                       preferred_element_type=jnp.float32)
            supp_blk = supp_all.reshape(nbw, B)
            later = (blk_row > k)
            kill = (later & (supp_blk > 0.5)).astype(jnp.float32)
            keep_ref[cs:cs + nbw, :] = keep_ref[cs:cs + nbw, :] * (1.0 - kill)
            return 0

        return blk_body

    lax.fori_loop(0, jnp.minimum(5, nblk), make_blk_body(0, NB), 0)
    lax.fori_loop(5, jnp.minimum(10, nblk), make_blk_body(5, 35), 0)
    lax.fori_loop(10, jnp.minimum(15, nblk), make_blk_body(10, 30), 0)
    lax.fori_loop(15, jnp.minimum(20, nblk), make_blk_body(15, 25), 0)
    lax.fori_loop(20, jnp.minimum(25, nblk), make_blk_body(20, 20), 0)
    lax.fori_loop(25, jnp.minimum(30, nblk), make_blk_body(25, 15), 0)
    lax.fori_loop(30, jnp.minimum(35, nblk), make_blk_body(30, 10), 0)
    lax.fori_loop(35, jnp.minimum(NB, nblk), make_blk_body(35, 5), 0)

    s = s_ref[...]
    gated_ref[...] = keep_ref[...] * s * (s >= SCORE_THRESH).astype(jnp.float32)


def _nms_call(rows, cols, s_blk):
    return pl.pallas_call(
        _nms_body,
        out_shape=jax.ShapeDtypeStruct((NB, B), jnp.float32),
        scratch_shapes=[pltpu.VMEM((NB, B), jnp.float32),
                        pltpu.VMEM((8, NP), jnp.float32),
                        pltpu.VMEM((B, NP), jnp.float32)],
    )(rows, cols, s_blk)


@functools.cache
def _make_sc_gather():
    @functools.partial(
        pl.kernel,
        mesh=plsc.VectorSubcoreMesh(core_axis_name="c", subcore_axis_name="s"),
        out_type=jax.ShapeDtypeStruct((NP, D), jnp.float32),
        scratch_types=[
            pltpu.VMEM((_ROWS_PER_W,), jnp.int32),
            pltpu.VMEM((_ROWS_PER_W, D), jnp.float32),
            pltpu.SemaphoreType.DMA,
        ],
    )
    def _sc_gather(table_hbm, idx_hbm, out_hbm, idx_v, rows_v, sem):
        wid = lax.axis_index("s") * _SC_NC + lax.axis_index("c")
        base = wid * _ROWS_PER_W
        pltpu.sync_copy(idx_hbm.at[pl.ds(base, _ROWS_PER_W)], idx_v)
        pltpu.async_copy(table_hbm.at[idx_v], rows_v, sem).wait()
        pltpu.sync_copy(rows_v, out_hbm.at[pl.ds(base, _ROWS_PER_W)])

    return _sc_gather


def _scale_body(rows_ref, g_ref, out_ref):
    out_ref[...] = rows_ref[0:N, :] * g_ref[0:N, :]


def _scale_call(rows, g_col):
    return pl.pallas_call(
        _scale_body,
        out_shape=jax.ShapeDtypeStruct((N, D), jnp.float32),
    )(rows, g_col)


def kernel(boxes, scores, labels, hidden_states):
    f32 = jnp.float32
    order = jnp.argsort(-scores)
    b_s = boxes[order].astype(f32)
    s_s = scores[order].astype(f32)
    l_s = labels[order].astype(f32)

    pad = NP - N
    rows5 = jnp.concatenate([b_s, l_s[:, None]], axis=1)
    pad5 = jnp.concatenate(
        [jnp.zeros((pad, 4), f32), jnp.full((pad, 1), -1.0, f32)], axis=1)
    rows5 = jnp.concatenate([rows5, pad5], axis=0)
    rows8 = jnp.pad(rows5, ((0, 0), (0, 3)))
    cols8 = jnp.pad(rows5.T, ((0, 3), (0, 0)))
    s_p = jnp.concatenate([s_s, jnp.full((pad,), -1.0, f32)])
    s_blk = s_p.reshape(NB, B)

    gated = _nms_call(rows8, cols8, s_blk)
    g_col = gated.reshape(NP, 1)

    order_p = jnp.concatenate(
        [order.astype(jnp.int32), jnp.zeros((pad,), jnp.int32)])
    gathered = _make_sc_gather()(hidden_states.astype(f32), order_p)

    return _scale_call(gathered, g_col)


# speedup vs baseline: 1.9620x; 1.9620x over previous
import functools

import jax
import jax.numpy as jnp
from jax import lax
from jax.experimental import pallas as pl
from jax.experimental.pallas import tpu as pltpu
from jax.experimental.pallas import tpu_sc as plsc

N = 5000
D = 256
B = 128
NB = 40
NP = NB * B
IOU_THRESH = 0.5
SCORE_THRESH = 0.2

_SC_NC = 2
_SC_NS = 16
_SC_NW = _SC_NC * _SC_NS
_ROWS_PER_W = NP // _SC_NW


def _nms_body(rows_ref, cols_ref, s_ref, gated_ref, keep_ref, coff_ref):
    offf = jnp.max(rows_ref[:, 0:4]) + 1.0
    labc = cols_ref[4:5, :]
    offc = labc * offf
    x1c = cols_ref[0:1, :] + offc
    y1c = cols_ref[1:2, :] + offc
    x2c = cols_ref[2:3, :] + offc
    y2c = cols_ref[3:4, :] + offc
    coff_ref[0:1, :] = x1c
    coff_ref[1:2, :] = y1c
    coff_ref[2:3, :] = x2c
    coff_ref[3:4, :] = y2c
    area_c = (x2c - x1c) * (y2c - y1c)
    coff_ref[4:5, :] = area_c

    keep_ref[...] = jnp.ones((NB, B), jnp.float32)

    n_above = jnp.sum((s_ref[...] >= SCORE_THRESH).astype(jnp.int32))
    nblk = (n_above + (B - 1)) // B

    tri_r = lax.broadcasted_iota(jnp.int32, (B, B), 0)
    tri_c = lax.broadcasted_iota(jnp.int32, (B, B), 1)
    trif = (tri_r < tri_c).astype(jnp.float32)

    def make_blk_body(cs, nbw):
        c0 = cs * B
        cw = nbw * B
        xc1 = coff_ref[0:1, c0:c0 + cw]
        yc1 = coff_ref[1:2, c0:c0 + cw]
        xc2 = coff_ref[2:3, c0:c0 + cw]
        yc2 = coff_ref[3:4, c0:c0 + cw]
        ac = coff_ref[4:5, c0:c0 + cw]
        blk_row = lax.broadcasted_iota(jnp.int32, (nbw, B), 0) + cs

        def blk_body(k, _):
            base = pl.multiple_of(k * B, B)
            r = rows_ref[pl.ds(base, B), :]
            offr = r[:, 4:5] * offf
            x1r = r[:, 0:1] + offr
            y1r = r[:, 1:2] + offr
            x2r = r[:, 2:3] + offr
            y2r = r[:, 3:4] + offr
            area_r = (x2r - x1r) * (y2r - y1r)

            iw = jnp.maximum(jnp.minimum(x2r, xc2) - jnp.maximum(x1r, xc1), 0.0)
            ih = jnp.minimum(y2r, yc2) - jnp.maximum(y1r, yc1)
            inter = iw * ih
            union = area_r + ac - inter
            mf = (inter > IOU_THRESH * union).astype(jnp.float32)

            x1b = coff_ref[0:1, pl.ds(base, B)]
            y1b = coff_ref[1:2, pl.ds(base, B)]
            x2b = coff_ref[2:3, pl.ds(base, B)]
            y2b = coff_ref[3:4, pl.ds(base, B)]
            area_b = coff_ref[4:5, pl.ds(base, B)]
            iwb = jnp.maximum(jnp.minimum(x2r, x2b) - jnp.maximum(x1r, x1b), 0.0)
            ihb = jnp.minimum(y2r, y2b) - jnp.maximum(y1r, y1b)
            interb = iwb * ihb
            unionb = area_r + area_b - interb
            mbf = (interb > IOU_THRESH * unionb).astype(jnp.float32) * trif

            init = keep_ref[pl.ds(k, 1), :]

            def step(kb):
                supp = lax.dot_general(kb, mbf, (((1,), (0,)), ((), ())),
                                       preferred_element_type=jnp.float32)
                return init * (supp < 0.5).astype(jnp.float32)

            def wcond(c):
                return c[1]

            def wbody(c):
                kb, _ = c
                kb1 = step(kb)
                kb2 = step(kb1)
                return (kb2, jnp.any(kb2 != kb1))

            kb, _ = lax.while_loop(wcond, wbody, (init, True))
            keep_ref[pl.ds(k, 1), :] = kb

            supp_all = lax.dot_general(kb, mf, (((1,), (0,)), ((), ())),
                                       preferred_element_type=jnp.float32)
            supp_blk = supp_all.reshape(nbw, B)
            later = (blk_row > k)
            kill = (later & (supp_blk > 0.5)).astype(jnp.float32)
            keep_ref[cs:cs + nbw, :] = keep_ref[cs:cs + nbw, :] * (1.0 - kill)
            return 0

        return blk_body

    lax.fori_loop(0, jnp.minimum(5, nblk), make_blk_body(0, NB), 0)
    lax.fori_loop(5, jnp.minimum(10, nblk), make_blk_body(5, 35), 0)
    lax.fori_loop(10, jnp.minimum(15, nblk), make_blk_body(10, 30), 0)
    lax.fori_loop(15, jnp.minimum(20, nblk), make_blk_body(15, 25), 0)
    lax.fori_loop(20, jnp.minimum(25, nblk), make_blk_body(20, 20), 0)
    lax.fori_loop(25, jnp.minimum(30, nblk), make_blk_body(25, 15), 0)
    lax.fori_loop(30, jnp.minimum(35, nblk), make_blk_body(30, 10), 0)
    lax.fori_loop(35, jnp.minimum(NB, nblk), make_blk_body(35, 5), 0)

    s = s_ref[...]
    gated_ref[...] = keep_ref[...] * s * (s >= SCORE_THRESH).astype(jnp.float32)


def _nms_call(rows, cols, s_blk):
    return pl.pallas_call(
        _nms_body,
        out_shape=jax.ShapeDtypeStruct((NB, B), jnp.float32),
        scratch_shapes=[pltpu.VMEM((NB, B), jnp.float32),
                        pltpu.VMEM((8, NP), jnp.float32)],
    )(rows, cols, s_blk)


@functools.cache
def _make_sc_gather():
    @functools.partial(
        pl.kernel,
        mesh=plsc.VectorSubcoreMesh(core_axis_name="c", subcore_axis_name="s"),
        out_type=jax.ShapeDtypeStruct((NP, D), jnp.float32),
        scratch_types=[
            pltpu.VMEM((_ROWS_PER_W,), jnp.int32),
            pltpu.VMEM((_ROWS_PER_W, D), jnp.float32),
            pltpu.SemaphoreType.DMA,
        ],
    )
    def _sc_gather(table_hbm, idx_hbm, out_hbm, idx_v, rows_v, sem):
        wid = lax.axis_index("s") * _SC_NC + lax.axis_index("c")
        base = wid * _ROWS_PER_W
        pltpu.sync_copy(idx_hbm.at[pl.ds(base, _ROWS_PER_W)], idx_v)
        pltpu.async_copy(table_hbm.at[idx_v], rows_v, sem).wait()
        pltpu.sync_copy(rows_v, out_hbm.at[pl.ds(base, _ROWS_PER_W)])

    return _sc_gather


def _scale_body(rows_ref, g_ref, out_ref):
    out_ref[...] = rows_ref[0:N, :] * g_ref[0:N, :]


def _scale_call(rows, g_col):
    return pl.pallas_call(
        _scale_body,
        out_shape=jax.ShapeDtypeStruct((N, D), jnp.float32),
    )(rows, g_col)


def kernel(boxes, scores, labels, hidden_states):
    f32 = jnp.float32
    order = jnp.argsort(-scores)
    b_s = boxes[order].astype(f32)
    s_s = scores[order].astype(f32)
    l_s = labels[order].astype(f32)

    pad = NP - N
    rows5 = jnp.concatenate([b_s, l_s[:, None]], axis=1)
    pad5 = jnp.concatenate(
        [jnp.zeros((pad, 4), f32), jnp.full((pad, 1), -1.0, f32)], axis=1)
    rows5 = jnp.concatenate([rows5, pad5], axis=0)
    rows8 = jnp.pad(rows5, ((0, 0), (0, 3)))
    cols8 = jnp.pad(rows5.T, ((0, 3), (0, 0)))
    s_p = jnp.concatenate([s_s, jnp.full((pad,), -1.0, f32)])
    s_blk = s_p.reshape(NB, B)

    gated = _nms_call(rows8, cols8, s_blk)
    g_col = gated.reshape(NP, 1)

    order_p = jnp.concatenate(
        [order.astype(jnp.int32), jnp.zeros((pad,), jnp.int32)])
    gathered = _make_sc_gather()(hidden_states.astype(f32), order_p)

    return _scale_call(gathered, g_col)
